# baseline (speedup 1.0000x reference)
import jax
import jax.numpy as jnp
from jax.experimental import pallas as pl

_N = 1024
_CUTOFF = 0.5
_INFINITY = 1000000.0
_MARGIN = 0.2
_NU = 0.0
_MAXF32BITS = 0x7F7FFFFF


def _margin_body(e_ref, tc_ref, tr_ref, beta_ref, out_ref):
    n = _N
    f32 = jnp.float32
    e = e_ref[:, :]
    t_col = tc_ref[:, :]
    t_row = tr_ref[:, :]
    beta = beta_ref[0, 0]

    ii = jax.lax.broadcasted_iota(jnp.int32, (n, n), 0)
    jj = jax.lax.broadcasted_iota(jnp.int32, (n, n), 1)
    eye = ii == jj

    prod = jax.lax.dot_general(
        e, e, (((1,), (1,)), ((), ())),
        precision=jax.lax.Precision.HIGHEST,
        preferred_element_type=f32)
    norm_col = jnp.sum(e * e, axis=1, keepdims=True)
    norm_row = jnp.sum(
        jnp.where(eye, prod, 0.0), axis=0, keepdims=True)
    res = norm_col + norm_row - 2.0 * prod
    q = jnp.maximum(res, 0.0)
    dl = jnp.sqrt(q + 1e-8)

    pos = (t_col == t_row) & ~eye
    posf = pos.astype(f32)
    c = jnp.sum(posf, axis=1, keepdims=True)
    s_total = jnp.sum(c)
    num_neg = jnp.floor(s_total / n)
    k = jnp.maximum(1.0, jnp.minimum(num_neg, float(n)))
    m = jnp.minimum(jnp.minimum(num_neg, c), k)

    d = jnp.sqrt(jnp.maximum(q, 1e-4))
    maskbit = pos | (d < _CUTOFF)
    maskf = maskbit.astype(f32)
    masked = d + _INFINITY * maskf
    bits = jax.lax.bitcast_convert_type(masked, jnp.int32)

    def _bs_val(_, lh):
        lo, hi, clo = lh
        mid = lo + jax.lax.shift_right_logical(hi - lo, 1)
        cnt = jnp.sum((bits <= mid).astype(f32), axis=1, keepdims=True)
        ge = cnt >= k
        return (jnp.where(ge, lo, mid + 1), jnp.where(ge, mid, hi),
                jnp.where(ge, clo, cnt))

    n_unmasked = float(n) - jnp.sum(maskf, axis=1, keepdims=True)
    unm_max = jnp.max(jnp.where(maskbit, 0, bits), axis=1, keepdims=True)
    lo0 = jnp.min(bits, axis=1, keepdims=True) - 1
    hi0 = jnp.where(n_unmasked >= k, unm_max, _MAXF32BITS)
    span = jnp.max((hi0 - lo0).astype(f32))
    trips_v = jnp.minimum(
        31.0, jnp.floor(jnp.log2(jnp.maximum(span, 1.0) + 1.0)) + 2.0
    ).astype(jnp.int32)
    _, thr, cnt_lt = jax.lax.fori_loop(
        0, trips_v, _bs_val, (lo0, hi0, jnp.zeros((n, 1), f32)))

    r_tie = k - cnt_lt
    tie = bits == thr
    tcount = jnp.sum(tie.astype(f32), axis=1, keepdims=True)

    def _bs_idx(_, lh):
        lo, hi = lh
        mid = lo + jax.lax.div(hi - lo, 2)
        cnt = jnp.sum((tie & (jj <= mid)).astype(f32), axis=1, keepdims=True)
        ge = cnt >= r_tie
        return jnp.where(ge, lo, mid + 1), jnp.where(ge, mid, hi)

    need_tie = jnp.sum((r_tie < tcount).astype(f32)) > 0.0
    trips_t = jnp.where(need_tie, 10, 0)
    lo1 = jnp.zeros((n, 1), jnp.int32)
    hi1 = jnp.full((n, 1), n - 1, jnp.int32)
    _, ucut = jax.lax.fori_loop(0, trips_t, _bs_idx, (lo1, hi1))
    sel = (bits < thr) | (tie & (jj <= ucut))
    self_f = sel.astype(f32)

    bf16 = jnp.bfloat16
    upper = (ii <= jj).astype(bf16)
    cpos = jax.lax.dot_general(
        posf.astype(bf16), upper, (((1,), (0,)), ((), ())),
        preferred_element_type=f32)
    cneg = jax.lax.dot_general(
        self_f.astype(bf16), upper, (((1,), (0,)), ((), ())),
        preferred_element_type=f32)
    uposf = posf * (cpos <= m).astype(f32)
    unegf = self_f * (cneg <= m).astype(f32)

    pterm = jnp.maximum(dl - beta + _MARGIN, 0.0)
    nterm = jnp.maximum(beta - dl + _MARGIN, 0.0)
    pos_sum = jnp.sum(uposf * pterm)
    neg_sum = jnp.sum(unegf * nterm)

    not_a = (pterm <= 0.0).astype(f32)
    not_b = (nterm <= 0.0).astype(f32)
    za = jnp.sum(uposf * not_a, axis=1, keepdims=True)
    zb = jnp.sum(unegf * not_b, axis=1, keepdims=True)

    amb = (za > 0.0) & (zb > 0.0) & (za < m) & (zb < m)
    any_amb = jnp.sum(amb.astype(f32)) > 0.0

    def _ov_body(r, ov):
        rr = (r + 1).astype(f32)
        arow = jnp.sum(uposf * (cpos == rr).astype(f32) * not_a,
                       axis=1, keepdims=True)
        brow = jnp.sum(unegf * (cneg == rr).astype(f32) * not_b,
                       axis=1, keepdims=True)
        return ov + arow * brow

    trips = jnp.where(any_amb, k, 0.0).astype(jnp.int32)
    ov_loop = jax.lax.fori_loop(0, trips, _ov_body,
                                jnp.zeros((n, 1), f32))
    overlap = jnp.where(
        amb, ov_loop,
        jnp.where((za <= 0.0) | (zb <= 0.0), 0.0,
                  jnp.where(zb >= m, za, jnp.where(za >= m, zb, 0.0))))

    pair_cnt = jnp.sum(m) - jnp.sum(overlap)
    total = pos_sum + neg_sum
    beta_reg = jnp.abs(beta) * _NU
    loss = jnp.where(pair_cnt > 0.0,
                     (total + beta_reg) / jnp.maximum(pair_cnt, 1.0),
                     total)
    out_ref[:, :] = jnp.broadcast_to(loss, (1, 1))


def kernel(E, T, beta):
    t = T.astype(jnp.int32)
    out = pl.pallas_call(
        _margin_body,
        out_shape=jax.ShapeDtypeStruct((1, 1), jnp.float32),
    )(E, t.reshape(_N, 1), t.reshape(1, _N), beta.reshape(1, 1))
    return out.reshape(())


# speedup vs baseline: 1.0505x; 1.0505x over previous
import jax
import jax.numpy as jnp
from jax.experimental import pallas as pl

_N = 1024
_CUTOFF = 0.5
_INFINITY = 1000000.0
_MARGIN = 0.2
_NU = 0.0
_MAXF32BITS = 0x7F7FFFFF


def _margin_body(e_ref, tc_ref, tr_ref, beta_ref, out_ref):
    n = _N
    f32 = jnp.float32
    e = e_ref[:, :]
    t_col = tc_ref[:, :]
    t_row = tr_ref[:, :]
    beta = beta_ref[0, 0]

    ii = jax.lax.broadcasted_iota(jnp.int32, (n, n), 0)
    jj = jax.lax.broadcasted_iota(jnp.int32, (n, n), 1)
    eye = ii == jj

    prod = jax.lax.dot_general(
        e, e, (((1,), (1,)), ((), ())),
        precision=jax.lax.Precision.HIGHEST,
        preferred_element_type=f32)
    norm_col = jnp.sum(e * e, axis=1, keepdims=True)
    norm_row = jnp.sum(
        jnp.where(eye, prod, 0.0), axis=0, keepdims=True)
    res = norm_col + norm_row - 2.0 * prod
    q = jnp.maximum(res, 0.0)
    dl = jnp.sqrt(q + 1e-8)

    pos = (t_col == t_row) & ~eye
    posf = pos.astype(f32)
    c = jnp.sum(posf, axis=1, keepdims=True)
    s_total = jnp.sum(c)
    num_neg = jnp.floor(s_total / n)
    k = jnp.maximum(1.0, jnp.minimum(num_neg, float(n)))
    m = jnp.minimum(jnp.minimum(num_neg, c), k)

    d = jnp.sqrt(jnp.maximum(q, 1e-4))
    maskbit = pos | (d < _CUTOFF)
    maskf = maskbit.astype(f32)
    masked = d + _INFINITY * maskf
    bits = jax.lax.bitcast_convert_type(masked, jnp.int32)

    def _bs_val(_, lh):
        lo, hi = lh
        mid = lo + jax.lax.shift_right_logical(hi - lo, 1)
        cnt = jnp.sum((bits <= mid).astype(f32), axis=1, keepdims=True)
        ge = cnt >= k
        return jnp.where(ge, lo, mid + 1), jnp.where(ge, mid, hi)

    n_unmasked = float(n) - jnp.sum(maskf, axis=1, keepdims=True)
    unm_max = jnp.max(jnp.where(maskbit, 0, bits), axis=1, keepdims=True)
    lo0 = jnp.min(bits, axis=1, keepdims=True) - 1
    hi0 = jnp.where(n_unmasked >= k, unm_max, _MAXF32BITS)
    span = jnp.max((hi0 - lo0).astype(f32))
    trips_v = jnp.minimum(
        31.0, jnp.floor(jnp.log2(jnp.maximum(span, 1.0) + 1.0)) + 2.0
    ).astype(jnp.int32)
    _, thr = jax.lax.fori_loop(0, trips_v, _bs_val, (lo0, hi0))

    cnt_lt = jnp.sum((bits < thr).astype(f32), axis=1, keepdims=True)
    r_tie = k - cnt_lt
    tie = bits == thr
    tcount = jnp.sum(tie.astype(f32), axis=1, keepdims=True)

    def _bs_idx(_, lh):
        lo, hi = lh
        mid = lo + jax.lax.div(hi - lo, 2)
        cnt = jnp.sum((tie & (jj <= mid)).astype(f32), axis=1, keepdims=True)
        ge = cnt >= r_tie
        return jnp.where(ge, lo, mid + 1), jnp.where(ge, mid, hi)

    need_tie = jnp.sum((r_tie < tcount).astype(f32)) > 0.0
    trips_t = jnp.where(need_tie, 10, 0)
    lo1 = jnp.zeros((n, 1), jnp.int32)
    hi1 = jnp.full((n, 1), n - 1, jnp.int32)
    _, ucut = jax.lax.fori_loop(0, trips_t, _bs_idx, (lo1, hi1))
    sel = (bits < thr) | (tie & (jj <= ucut))
    self_f = sel.astype(f32)

    bf16 = jnp.bfloat16
    upper = (ii <= jj).astype(bf16)
    cpos = jax.lax.dot_general(
        posf.astype(bf16), upper, (((1,), (0,)), ((), ())),
        preferred_element_type=f32)
    cneg = jax.lax.dot_general(
        self_f.astype(bf16), upper, (((1,), (0,)), ((), ())),
        preferred_element_type=f32)
    uposf = posf * (cpos <= m).astype(f32)
    unegf = self_f * (cneg <= m).astype(f32)

    pterm = jnp.maximum(dl - beta + _MARGIN, 0.0)
    nterm = jnp.maximum(beta - dl + _MARGIN, 0.0)
    pos_sum = jnp.sum(uposf * pterm)
    neg_sum = jnp.sum(unegf * nterm)

    not_a = (pterm <= 0.0).astype(f32)
    not_b = (nterm <= 0.0).astype(f32)
    za = jnp.sum(uposf * not_a, axis=1, keepdims=True)
    zb = jnp.sum(unegf * not_b, axis=1, keepdims=True)

    amb = (za > 0.0) & (zb > 0.0) & (za < m) & (zb < m)
    any_amb = jnp.sum(amb.astype(f32)) > 0.0

    def _ov_body(r, ov):
        rr = (r + 1).astype(f32)
        arow = jnp.sum(uposf * (cpos == rr).astype(f32) * not_a,
                       axis=1, keepdims=True)
        brow = jnp.sum(unegf * (cneg == rr).astype(f32) * not_b,
                       axis=1, keepdims=True)
        return ov + arow * brow

    trips = jnp.where(any_amb, k, 0.0).astype(jnp.int32)
    ov_loop = jax.lax.fori_loop(0, trips, _ov_body,
                                jnp.zeros((n, 1), f32))
    overlap = jnp.where(
        amb, ov_loop,
        jnp.where((za <= 0.0) | (zb <= 0.0), 0.0,
                  jnp.where(zb >= m, za, jnp.where(za >= m, zb, 0.0))))

    pair_cnt = jnp.sum(m) - jnp.sum(overlap)
    total = pos_sum + neg_sum
    beta_reg = jnp.abs(beta) * _NU
    loss = jnp.where(pair_cnt > 0.0,
                     (total + beta_reg) / jnp.maximum(pair_cnt, 1.0),
                     total)
    out_ref[:, :] = jnp.broadcast_to(loss, (1, 1))


def kernel(E, T, beta):
    t = T.astype(jnp.int32)
    out = pl.pallas_call(
        _margin_body,
        out_shape=jax.ShapeDtypeStruct((1, 1), jnp.float32),
    )(E, t.reshape(_N, 1), t.reshape(1, _N), beta.reshape(1, 1))
    return out.reshape(())


# speedup vs baseline: 1.1113x; 1.0579x over previous
import jax
import jax.numpy as jnp
from jax.experimental import pallas as pl

_N = 1024
_CUTOFF = 0.5
_INFINITY = 1000000.0
_MARGIN = 0.2
_NU = 0.0
_MAXF32BITS = 0x7F7FFFFF


def _margin_body(e_ref, tc_ref, tr_ref, beta_ref, out_ref):
    n = _N
    f32 = jnp.float32
    e = e_ref[:, :]
    t_col = tc_ref[:, :]
    t_row = tr_ref[:, :]
    beta = beta_ref[0, 0]

    ii = jax.lax.broadcasted_iota(jnp.int32, (n, n), 0)
    jj = jax.lax.broadcasted_iota(jnp.int32, (n, n), 1)
    eye = ii == jj

    prod = jax.lax.dot_general(
        e, e, (((1,), (1,)), ((), ())),
        preferred_element_type=f32)
    norm_col = jnp.sum(e * e, axis=1, keepdims=True)
    norm_row = jnp.sum(
        jnp.where(eye, prod, 0.0), axis=0, keepdims=True)
    res = norm_col + norm_row - 2.0 * prod
    q = jnp.maximum(res, 0.0)
    dl = jnp.sqrt(q + 1e-8)

    pos = (t_col == t_row) & ~eye
    posf = pos.astype(f32)
    c = jnp.sum(posf, axis=1, keepdims=True)
    s_total = jnp.sum(c)
    num_neg = jnp.floor(s_total / n)
    k = jnp.maximum(1.0, jnp.minimum(num_neg, float(n)))
    m = jnp.minimum(jnp.minimum(num_neg, c), k)

    d = jnp.sqrt(jnp.maximum(q, 1e-4))
    maskbit = pos | (d < _CUTOFF)
    maskf = maskbit.astype(f32)
    masked = d + _INFINITY * maskf
    bits = jax.lax.bitcast_convert_type(masked, jnp.int32)

    def _bs_val(_, lh):
        lo, hi = lh
        mid = lo + jax.lax.shift_right_logical(hi - lo, 1)
        cnt = jnp.sum((bits <= mid).astype(f32), axis=1, keepdims=True)
        ge = cnt >= k
        return jnp.where(ge, lo, mid + 1), jnp.where(ge, mid, hi)

    n_unmasked = float(n) - jnp.sum(maskf, axis=1, keepdims=True)
    unm_max = jnp.max(jnp.where(maskbit, 0, bits), axis=1, keepdims=True)
    lo0 = jnp.min(bits, axis=1, keepdims=True) - 1
    hi0 = jnp.where(n_unmasked >= k, unm_max, _MAXF32BITS)
    span = jnp.max((hi0 - lo0).astype(f32))
    trips_v = jnp.minimum(
        31.0, jnp.floor(jnp.log2(jnp.maximum(span, 1.0) + 1.0)) + 2.0
    ).astype(jnp.int32)
    _, thr = jax.lax.fori_loop(0, trips_v, _bs_val, (lo0, hi0))

    cnt_lt = jnp.sum((bits < thr).astype(f32), axis=1, keepdims=True)
    r_tie = k - cnt_lt
    tie = bits == thr
    tcount = jnp.sum(tie.astype(f32), axis=1, keepdims=True)

    def _bs_idx(_, lh):
        lo, hi = lh
        mid = lo + jax.lax.div(hi - lo, 2)
        cnt = jnp.sum((tie & (jj <= mid)).astype(f32), axis=1, keepdims=True)
        ge = cnt >= r_tie
        return jnp.where(ge, lo, mid + 1), jnp.where(ge, mid, hi)

    need_tie = jnp.sum((r_tie < tcount).astype(f32)) > 0.0
    trips_t = jnp.where(need_tie, 10, 0)
    lo1 = jnp.zeros((n, 1), jnp.int32)
    hi1 = jnp.full((n, 1), n - 1, jnp.int32)
    _, ucut = jax.lax.fori_loop(0, trips_t, _bs_idx, (lo1, hi1))
    sel = (bits < thr) | (tie & (jj <= ucut))
    self_f = sel.astype(f32)

    bf16 = jnp.bfloat16
    upper = (ii <= jj).astype(bf16)
    cpos = jax.lax.dot_general(
        posf.astype(bf16), upper, (((1,), (0,)), ((), ())),
        preferred_element_type=f32)
    cneg = jax.lax.dot_general(
        self_f.astype(bf16), upper, (((1,), (0,)), ((), ())),
        preferred_element_type=f32)
    uposf = posf * (cpos <= m).astype(f32)
    unegf = self_f * (cneg <= m).astype(f32)

    pterm = jnp.maximum(dl - beta + _MARGIN, 0.0)
    nterm = jnp.maximum(beta - dl + _MARGIN, 0.0)
    pos_sum = jnp.sum(uposf * pterm)
    neg_sum = jnp.sum(unegf * nterm)

    not_a = (pterm <= 0.0).astype(f32)
    not_b = (nterm <= 0.0).astype(f32)
    za = jnp.sum(uposf * not_a, axis=1, keepdims=True)
    zb = jnp.sum(unegf * not_b, axis=1, keepdims=True)

    amb = (za > 0.0) & (zb > 0.0) & (za < m) & (zb < m)
    any_amb = jnp.sum(amb.astype(f32)) > 0.0

    def _ov_body(r, ov):
        rr = (r + 1).astype(f32)
        arow = jnp.sum(uposf * (cpos == rr).astype(f32) * not_a,
                       axis=1, keepdims=True)
        brow = jnp.sum(unegf * (cneg == rr).astype(f32) * not_b,
                       axis=1, keepdims=True)
        return ov + arow * brow

    trips = jnp.where(any_amb, k, 0.0).astype(jnp.int32)
    ov_loop = jax.lax.fori_loop(0, trips, _ov_body,
                                jnp.zeros((n, 1), f32))
    overlap = jnp.where(
        amb, ov_loop,
        jnp.where((za <= 0.0) | (zb <= 0.0), 0.0,
                  jnp.where(zb >= m, za, jnp.where(za >= m, zb, 0.0))))

    pair_cnt = jnp.sum(m) - jnp.sum(overlap)
    total = pos_sum + neg_sum
    beta_reg = jnp.abs(beta) * _NU
    loss = jnp.where(pair_cnt > 0.0,
                     (total + beta_reg) / jnp.maximum(pair_cnt, 1.0),
                     total)
    out_ref[:, :] = jnp.broadcast_to(loss, (1, 1))


def kernel(E, T, beta):
    t = T.astype(jnp.int32)
    out = pl.pallas_call(
        _margin_body,
        out_shape=jax.ShapeDtypeStruct((1, 1), jnp.float32),
    )(E, t.reshape(_N, 1), t.reshape(1, _N), beta.reshape(1, 1))
    return out.reshape(())
